# Initial kernel scaffold; baseline (speedup 1.0000x reference)
import jax
import jax.numpy as jnp
from jax.experimental import pallas as pl


def kernel(x, weight):
    raise NotImplementedError("write your pallas kernel here")


# speedup vs baseline: 1.8393x; 1.8393x over previous
import functools

import jax
import jax.numpy as jnp
from jax import lax
from jax.experimental import pallas as pl
from jax.experimental.pallas import tpu as pltpu
from jax.experimental.pallas import tpu_sc as plsc

VOCAB = 100000
D = 128
BATCH = 4096
HIST = 50

NC = 2
NS = 16
NW = NC * NS
ROWS = BATCH * HIST
R_PER_W = ROWS // NW
CHUNK = 128
NCHUNK = R_PER_W // CHUNK


def _rsqrt_newton(s):
    i = lax.bitcast_convert_type(s, jnp.int32)
    i = jnp.int32(0x5F3759DF) - (i >> 1)
    y = lax.bitcast_convert_type(i, jnp.float32)
    for _ in range(3):
        y = y * (1.5 - 0.5 * s * y * y)
    return y


def _sc_body(x_hbm, w_hbm, out_hbm, idx_v, buf, sem):
    c = lax.axis_index("c")
    s = lax.axis_index("s")
    wid = s * NC + c

    pltpu.sync_copy(x_hbm.at[wid], idx_v)

    def chunk_body(j, carry):
        pltpu.async_copy(w_hbm.at[idx_v.at[j]], buf, sem).wait()

        def row_body(r, carry2):
            vs = [buf[r, pl.ds(16 * k, 16)] for k in range(8)]
            ss = vs[0] * vs[0]
            for k in range(1, 8):
                ss = ss + vs[k] * vs[k]
            csum = plsc.cumsum(ss)
            tot = jnp.maximum(jnp.broadcast_to(csum[15], (16,)),
                              jnp.float32(1e-24))
            scale = _rsqrt_newton(tot)
            for k in range(8):
                buf[r, pl.ds(16 * k, 16)] = vs[k] * scale
            return carry2

        lax.fori_loop(0, CHUNK, row_body, 0, unroll=False)

        base = (wid * NCHUNK + j) * CHUNK
        pltpu.sync_copy(buf, out_hbm.at[pl.ds(base, CHUNK)])
        return carry

    lax.fori_loop(0, NCHUNK, chunk_body, 0, unroll=False)


@jax.jit
def kernel(x, weight):
    xi = x.astype(jnp.int32).reshape(NW, NCHUNK, CHUNK)
    mesh = plsc.VectorSubcoreMesh(core_axis_name="c", subcore_axis_name="s")
    out = pl.kernel(
        _sc_body,
        out_type=jax.ShapeDtypeStruct((ROWS, D), jnp.float32),
        mesh=mesh,
        compiler_params=pltpu.CompilerParams(needs_layout_passes=False),
        scratch_types=[
            pltpu.VMEM((NCHUNK, CHUNK), jnp.int32),
            pltpu.VMEM((CHUNK, D), jnp.float32),
            pltpu.SemaphoreType.DMA,
        ],
    )(xi, weight)
    return out.reshape(BATCH, HIST, D)


# speedup vs baseline: 2.1188x; 1.1520x over previous
import functools

import jax
import jax.numpy as jnp
from jax import lax
from jax.experimental import pallas as pl
from jax.experimental.pallas import tpu as pltpu
from jax.experimental.pallas import tpu_sc as plsc

VOCAB = 100000
D = 128
BATCH = 4096
HIST = 50

NC = 2
NS = 16
NW = NC * NS
ROWS = BATCH * HIST
R_PER_W = ROWS // NW
CHUNK = 128
NCHUNK = R_PER_W // CHUNK


def _rsqrt_newton(s):
    i = lax.bitcast_convert_type(s, jnp.int32)
    i = jnp.int32(0x5F3759DF) - (i >> 1)
    y = lax.bitcast_convert_type(i, jnp.float32)
    for _ in range(3):
        y = y * (1.5 - 0.5 * s * y * y)
    return y


def _normalize_chunk(buf):
    def row_body(r, carry):
        vs = [buf[r, pl.ds(16 * k, 16)] for k in range(8)]
        ss = vs[0] * vs[0]
        for k in range(1, 8):
            ss = ss + vs[k] * vs[k]
        csum = plsc.cumsum(ss)
        tot = jnp.maximum(jnp.broadcast_to(csum[15], (16,)),
                          jnp.float32(1e-24))
        scale = _rsqrt_newton(tot)
        for k in range(8):
            buf[r, pl.ds(16 * k, 16)] = vs[k] * scale
        return carry

    lax.fori_loop(0, CHUNK, row_body, 0, unroll=False)


def _sc_body(x_hbm, w_hbm, out_hbm, idx_v, buf0, buf1, g0, g1, st0, st1):
    c = lax.axis_index("c")
    s = lax.axis_index("s")
    wid = s * NC + c

    pltpu.sync_copy(x_hbm.at[wid], idx_v)

    def gather(j, buf, sem):
        return pltpu.async_copy(w_hbm.at[idx_v.at[j]], buf, sem)

    def wait_gather(j, buf, sem):
        pltpu.make_async_copy(w_hbm.at[idx_v.at[j]], buf, sem).wait()

    def store(j, buf, sem):
        base = (wid * NCHUNK + j) * CHUNK
        return pltpu.async_copy(buf, out_hbm.at[pl.ds(base, CHUNK)], sem)

    gather(0, buf0, g0)
    gather(1, buf1, g1)

    def body(i, carry):
        j0 = 2 * i
        wait_gather(j0, buf0, g0)
        _normalize_chunk(buf0)
        h0 = store(j0, buf0, st0)
        wait_gather(j0 + 1, buf1, g1)
        h0.wait()
        gather(j0 + 2, buf0, g0)
        _normalize_chunk(buf1)
        h1 = store(j0 + 1, buf1, st1)
        h1.wait()
        gather(j0 + 3, buf1, g1)
        return carry

    lax.fori_loop(0, (NCHUNK - 2) // 2, body, 0, unroll=False)

    wait_gather(NCHUNK - 2, buf0, g0)
    _normalize_chunk(buf0)
    h0 = store(NCHUNK - 2, buf0, st0)
    wait_gather(NCHUNK - 1, buf1, g1)
    _normalize_chunk(buf1)
    h1 = store(NCHUNK - 1, buf1, st1)
    h0.wait()
    h1.wait()


@jax.jit
def kernel(x, weight):
    xi = x.astype(jnp.int32).reshape(NW, NCHUNK, CHUNK)
    mesh = plsc.VectorSubcoreMesh(core_axis_name="c", subcore_axis_name="s")
    out = pl.kernel(
        _sc_body,
        out_type=jax.ShapeDtypeStruct((ROWS, D), jnp.float32),
        mesh=mesh,
        compiler_params=pltpu.CompilerParams(needs_layout_passes=False),
        scratch_types=[
            pltpu.VMEM((NCHUNK, CHUNK), jnp.int32),
            pltpu.VMEM((CHUNK, D), jnp.float32),
            pltpu.VMEM((CHUNK, D), jnp.float32),
            pltpu.SemaphoreType.DMA,
            pltpu.SemaphoreType.DMA,
            pltpu.SemaphoreType.DMA,
            pltpu.SemaphoreType.DMA,
        ],
    )(xi, weight)
    return out.reshape(BATCH, HIST, D)


# speedup vs baseline: 2.9620x; 1.3980x over previous
import functools

import jax
import jax.numpy as jnp
from jax import lax
from jax.experimental import pallas as pl
from jax.experimental.pallas import tpu as pltpu
from jax.experimental.pallas import tpu_sc as plsc

VOCAB = 100000
D = 128
BATCH = 4096
HIST = 50

NC = 2
NS = 16
NW = NC * NS
ROWS = BATCH * HIST
R_PER_W = ROWS // NW
CHUNK = 128
NCHUNK = R_PER_W // CHUNK

TC_BLOCK = 2000


def _tc_norm_body(w_ref, o_ref):
    w = w_ref[...]
    ss = jnp.sum(w * w, axis=1, keepdims=True)
    o_ref[...] = w / jnp.maximum(jnp.sqrt(ss), jnp.float32(1e-12))


def _normalize_table(weight):
    return pl.pallas_call(
        _tc_norm_body,
        out_shape=jax.ShapeDtypeStruct((VOCAB, D), jnp.float32),
        grid=(VOCAB // TC_BLOCK,),
        in_specs=[pl.BlockSpec((TC_BLOCK, D), lambda i: (i, 0))],
        out_specs=pl.BlockSpec((TC_BLOCK, D), lambda i: (i, 0)),
    )(weight)


def _sc_body(x_hbm, w_hbm, out_hbm, idx_v, buf0, buf1, g0, g1, st0, st1):
    c = lax.axis_index("c")
    s = lax.axis_index("s")
    wid = s * NC + c

    pltpu.sync_copy(x_hbm.at[wid], idx_v)

    def gather(j, buf, sem):
        return pltpu.async_copy(w_hbm.at[idx_v.at[j]], buf, sem)

    def wait_gather(j, buf, sem):
        pltpu.make_async_copy(w_hbm.at[idx_v.at[j]], buf, sem).wait()

    def store(j, buf, sem):
        base = (wid * NCHUNK + j) * CHUNK
        return pltpu.async_copy(buf, out_hbm.at[pl.ds(base, CHUNK)], sem)

    gather(0, buf0, g0)
    gather(1, buf1, g1)

    def body(i, carry):
        j0 = 2 * i
        wait_gather(j0, buf0, g0)
        h0 = store(j0, buf0, st0)
        wait_gather(j0 + 1, buf1, g1)
        h0.wait()
        gather(j0 + 2, buf0, g0)
        h1 = store(j0 + 1, buf1, st1)
        h1.wait()
        gather(j0 + 3, buf1, g1)
        return carry

    lax.fori_loop(0, (NCHUNK - 2) // 2, body, 0, unroll=False)

    wait_gather(NCHUNK - 2, buf0, g0)
    h0 = store(NCHUNK - 2, buf0, st0)
    wait_gather(NCHUNK - 1, buf1, g1)
    h1 = store(NCHUNK - 1, buf1, st1)
    h0.wait()
    h1.wait()


@jax.jit
def kernel(x, weight):
    wn = _normalize_table(weight)
    xi = x.astype(jnp.int32).reshape(NW, NCHUNK, CHUNK)
    mesh = plsc.VectorSubcoreMesh(core_axis_name="c", subcore_axis_name="s")
    out = pl.kernel(
        _sc_body,
        out_type=jax.ShapeDtypeStruct((ROWS, D), jnp.float32),
        mesh=mesh,
        compiler_params=pltpu.CompilerParams(needs_layout_passes=False),
        scratch_types=[
            pltpu.VMEM((NCHUNK, CHUNK), jnp.int32),
            pltpu.VMEM((CHUNK, D), jnp.float32),
            pltpu.VMEM((CHUNK, D), jnp.float32),
            pltpu.SemaphoreType.DMA,
            pltpu.SemaphoreType.DMA,
            pltpu.SemaphoreType.DMA,
            pltpu.SemaphoreType.DMA,
        ],
    )(xi, wn)
    return out.reshape(BATCH, HIST, D)


# speedup vs baseline: 7.6653x; 2.5879x over previous
import functools

import jax
import jax.numpy as jnp
from jax import lax
from jax.experimental import pallas as pl
from jax.experimental.pallas import tpu as pltpu
from jax.experimental.pallas import tpu_sc as plsc

VOCAB = 100000
D = 128
BATCH = 4096
HIST = 50

NC = 2
NS = 16
NW = NC * NS
ROWS = BATCH * HIST
R_PER_W = ROWS // NW
CHUNK = 128
NCHUNK = R_PER_W // CHUNK
NSLOT = 5

TC_BLOCK = 5000


def _tc_norm_body(w_ref, o_ref):
    w = w_ref[...]
    ss = jnp.sum(w * w, axis=1, keepdims=True)
    o_ref[...] = w / jnp.maximum(jnp.sqrt(ss), jnp.float32(1e-12))


def _normalize_table(weight):
    return pl.pallas_call(
        _tc_norm_body,
        out_shape=jax.ShapeDtypeStruct((VOCAB, D), jnp.float32),
        grid=(VOCAB // TC_BLOCK,),
        in_specs=[pl.BlockSpec((TC_BLOCK, D), lambda i: (i, 0))],
        out_specs=pl.BlockSpec((TC_BLOCK, D), lambda i: (i, 0)),
    )(weight)


def _sc_body(x_hbm, w_hbm, out_hbm, idx_v, *bufs_and_sems):
    bufs = bufs_and_sems[:NSLOT]
    gsem = bufs_and_sems[NSLOT:2 * NSLOT]
    ssem = bufs_and_sems[2 * NSLOT:3 * NSLOT]

    c = lax.axis_index("c")
    s = lax.axis_index("s")
    wid = s * NC + c

    pltpu.sync_copy(x_hbm.at[wid], idx_v)

    def gather(j, buf, sem):
        return pltpu.async_copy(w_hbm.at[idx_v.at[j]], buf, sem)

    def wait_gather(j, buf, sem):
        pltpu.make_async_copy(w_hbm.at[idx_v.at[j]], buf, sem).wait()

    def store(j, buf, sem):
        base = (wid * NCHUNK + j) * CHUNK
        return pltpu.async_copy(buf, out_hbm.at[pl.ds(base, CHUNK)], sem)

    def wait_store(j, buf, sem):
        base = (wid * NCHUNK + j) * CHUNK
        pltpu.make_async_copy(buf, out_hbm.at[pl.ds(base, CHUNK)], sem).wait()

    for k in range(NSLOT):
        gather(k, bufs[k], gsem[k])

    def body(i, carry):
        j0 = NSLOT * i
        for k in range(NSLOT):
            wait_gather(j0 + k, bufs[k], gsem[k])
            store(j0 + k, bufs[k], ssem[k])
        for k in range(NSLOT):
            wait_store(j0 + k, bufs[k], ssem[k])
            gather(j0 + NSLOT + k, bufs[k], gsem[k])
        return carry

    lax.fori_loop(0, NCHUNK // NSLOT - 1, body, 0, unroll=False)

    j0 = NCHUNK - NSLOT
    hs = []
    for k in range(NSLOT):
        wait_gather(j0 + k, bufs[k], gsem[k])
        hs.append(store(j0 + k, bufs[k], ssem[k]))
    for h in hs:
        h.wait()


@jax.jit
def kernel(x, weight):
    wn = _normalize_table(weight)
    xi = x.astype(jnp.int32).T.reshape(NW, NCHUNK, CHUNK)
    mesh = plsc.VectorSubcoreMesh(core_axis_name="c", subcore_axis_name="s")
    out = pl.kernel(
        _sc_body,
        out_type=jax.ShapeDtypeStruct((ROWS, D), jnp.float32),
        mesh=mesh,
        compiler_params=pltpu.CompilerParams(needs_layout_passes=False),
        scratch_types=(
            [pltpu.VMEM((NCHUNK, CHUNK), jnp.int32)]
            + [pltpu.VMEM((CHUNK, D), jnp.float32) for _ in range(NSLOT)]
            + [pltpu.SemaphoreType.DMA for _ in range(2 * NSLOT)]
        ),
    )(xi, wn)
    return out.reshape(HIST, BATCH, D).swapaxes(0, 1)


# speedup vs baseline: 7.8484x; 1.0239x over previous
import functools

import jax
import jax.numpy as jnp
from jax import lax
from jax.experimental import pallas as pl
from jax.experimental.pallas import tpu as pltpu
from jax.experimental.pallas import tpu_sc as plsc

VOCAB = 100000
D = 128
BATCH = 4096
HIST = 50

NC = 2
NS = 16
NW = NC * NS
ROWS = BATCH * HIST
R_PER_W = ROWS // NW
CHUNK = 128
NCHUNK = R_PER_W // CHUNK
NSLOT = 5

TC_BLOCK = 20000


def _tc_norm_body(w_ref, o_ref):
    w = w_ref[...]
    ss = jnp.sum(w * w, axis=1, keepdims=True)
    o_ref[...] = w / jnp.maximum(jnp.sqrt(ss), jnp.float32(1e-12))


def _normalize_table(weight):
    return pl.pallas_call(
        _tc_norm_body,
        out_shape=jax.ShapeDtypeStruct((VOCAB, D), jnp.float32),
        grid=(VOCAB // TC_BLOCK,),
        in_specs=[pl.BlockSpec((TC_BLOCK, D), lambda i: (i, 0))],
        out_specs=pl.BlockSpec((TC_BLOCK, D), lambda i: (i, 0)),
    )(weight)


def _sc_body(x_hbm, w_hbm, out_hbm, idx_v, *bufs_and_sems):
    bufs = bufs_and_sems[:NSLOT]
    gsem = bufs_and_sems[NSLOT:2 * NSLOT]
    ssem = bufs_and_sems[2 * NSLOT:3 * NSLOT]

    c = lax.axis_index("c")
    s = lax.axis_index("s")
    wid = s * NC + c

    pltpu.sync_copy(x_hbm.at[wid], idx_v)

    def gather(j, buf, sem):
        return pltpu.async_copy(w_hbm.at[idx_v.at[j]], buf, sem)

    def wait_gather(j, buf, sem):
        pltpu.make_async_copy(w_hbm.at[idx_v.at[j]], buf, sem).wait()

    def store(j, buf, sem):
        base = (wid * NCHUNK + j) * CHUNK
        return pltpu.async_copy(buf, out_hbm.at[pl.ds(base, CHUNK)], sem)

    def wait_store(j, buf, sem):
        base = (wid * NCHUNK + j) * CHUNK
        pltpu.make_async_copy(buf, out_hbm.at[pl.ds(base, CHUNK)], sem).wait()

    for k in range(NSLOT):
        gather(k, bufs[k], gsem[k])

    def body(i, carry):
        j0 = NSLOT * i
        for k in range(NSLOT):
            wait_gather(j0 + k, bufs[k], gsem[k])
            store(j0 + k, bufs[k], ssem[k])
        for k in range(NSLOT):
            wait_store(j0 + k, bufs[k], ssem[k])
            gather(j0 + NSLOT + k, bufs[k], gsem[k])
        return carry

    lax.fori_loop(0, NCHUNK // NSLOT - 1, body, 0, unroll=False)

    j0 = NCHUNK - NSLOT
    hs = []
    for k in range(NSLOT):
        wait_gather(j0 + k, bufs[k], gsem[k])
        hs.append(store(j0 + k, bufs[k], ssem[k]))
    for h in hs:
        h.wait()


@jax.jit
def kernel(x, weight):
    wn = _normalize_table(weight)
    xi = x.astype(jnp.int32).T.reshape(NW, NCHUNK, CHUNK)
    mesh = plsc.VectorSubcoreMesh(core_axis_name="c", subcore_axis_name="s")
    out = pl.kernel(
        _sc_body,
        out_type=jax.ShapeDtypeStruct((ROWS, D), jnp.float32),
        mesh=mesh,
        compiler_params=pltpu.CompilerParams(needs_layout_passes=False),
        scratch_types=(
            [pltpu.VMEM((NCHUNK, CHUNK), jnp.int32)]
            + [pltpu.VMEM((CHUNK, D), jnp.float32) for _ in range(NSLOT)]
            + [pltpu.SemaphoreType.DMA for _ in range(2 * NSLOT)]
        ),
    )(xi, wn)
    return out.reshape(HIST, BATCH, D).swapaxes(0, 1)


# speedup vs baseline: 7.8875x; 1.0050x over previous
import functools

import jax
import jax.numpy as jnp
from jax import lax
from jax.experimental import pallas as pl
from jax.experimental.pallas import tpu as pltpu
from jax.experimental.pallas import tpu_sc as plsc

VOCAB = 100000
D = 128
BATCH = 4096
HIST = 50

NC = 2
NS = 16
NW = NC * NS
ROWS = BATCH * HIST
R_PER_W = ROWS // NW
CHUNK = 128
NCHUNK = R_PER_W // CHUNK
NSLOT = 5

TC_BLOCK = 10000


def _tc_norm_body(w_ref, o_ref):
    w = w_ref[...]
    ss = jnp.sum(w * w, axis=1, keepdims=True)
    o_ref[...] = w / jnp.maximum(jnp.sqrt(ss), jnp.float32(1e-12))


def _normalize_table(weight):
    return pl.pallas_call(
        _tc_norm_body,
        out_shape=jax.ShapeDtypeStruct((VOCAB, D), jnp.float32),
        grid=(VOCAB // TC_BLOCK,),
        in_specs=[pl.BlockSpec((TC_BLOCK, D), lambda i: (i, 0))],
        out_specs=pl.BlockSpec((TC_BLOCK, D), lambda i: (i, 0)),
        compiler_params=pltpu.CompilerParams(
            dimension_semantics=("parallel",)
        ),
    )(weight)


def _sc_body(x_hbm, w_hbm, out_hbm, idx_v, *bufs_and_sems):
    bufs = bufs_and_sems[:NSLOT]
    gsem = bufs_and_sems[NSLOT:2 * NSLOT]
    ssem = bufs_and_sems[2 * NSLOT:3 * NSLOT]

    c = lax.axis_index("c")
    s = lax.axis_index("s")
    wid = s * NC + c

    pltpu.sync_copy(x_hbm.at[wid], idx_v)

    def gather(j, buf, sem):
        return pltpu.async_copy(w_hbm.at[idx_v.at[j]], buf, sem)

    def wait_gather(j, buf, sem):
        pltpu.make_async_copy(w_hbm.at[idx_v.at[j]], buf, sem).wait()

    def store(j, buf, sem):
        base = (wid * NCHUNK + j) * CHUNK
        return pltpu.async_copy(buf, out_hbm.at[pl.ds(base, CHUNK)], sem)

    def wait_store(j, buf, sem):
        base = (wid * NCHUNK + j) * CHUNK
        pltpu.make_async_copy(buf, out_hbm.at[pl.ds(base, CHUNK)], sem).wait()

    for k in range(NSLOT):
        gather(k, bufs[k], gsem[k])

    def body(i, carry):
        j0 = NSLOT * i
        for k in range(NSLOT):
            wait_gather(j0 + k, bufs[k], gsem[k])
            store(j0 + k, bufs[k], ssem[k])
        for k in range(NSLOT):
            wait_store(j0 + k, bufs[k], ssem[k])
            gather(j0 + NSLOT + k, bufs[k], gsem[k])
        return carry

    lax.fori_loop(0, NCHUNK // NSLOT - 1, body, 0, unroll=False)

    j0 = NCHUNK - NSLOT
    hs = []
    for k in range(NSLOT):
        wait_gather(j0 + k, bufs[k], gsem[k])
        hs.append(store(j0 + k, bufs[k], ssem[k]))
    for h in hs:
        h.wait()


@jax.jit
def kernel(x, weight):
    wn = _normalize_table(weight)
    xi = x.astype(jnp.int32).T.reshape(NW, NCHUNK, CHUNK)
    mesh = plsc.VectorSubcoreMesh(core_axis_name="c", subcore_axis_name="s")
    out = pl.kernel(
        _sc_body,
        out_type=jax.ShapeDtypeStruct((ROWS, D), jnp.float32),
        mesh=mesh,
        compiler_params=pltpu.CompilerParams(needs_layout_passes=False),
        scratch_types=(
            [pltpu.VMEM((NCHUNK, CHUNK), jnp.int32)]
            + [pltpu.VMEM((CHUNK, D), jnp.float32) for _ in range(NSLOT)]
            + [pltpu.SemaphoreType.DMA for _ in range(2 * NSLOT)]
        ),
    )(xi, wn)
    return out.reshape(HIST, BATCH, D).swapaxes(0, 1)


# speedup vs baseline: 7.9131x; 1.0032x over previous
import functools

import jax
import jax.numpy as jnp
from jax import lax
from jax.experimental import pallas as pl
from jax.experimental.pallas import tpu as pltpu
from jax.experimental.pallas import tpu_sc as plsc

VOCAB = 100000
D = 128
BATCH = 4096
HIST = 50

NC = 2
NS = 16
NW = NC * NS
ROWS = BATCH * HIST
R_PER_W = ROWS // NW
CHUNK = 128
NCHUNK = R_PER_W // CHUNK
NSLOT = 5

TC_BLOCK = 10000


def _tc_norm_body(w_ref, o_ref):
    w = w_ref[...]
    ss = jnp.sum(w * w, axis=1, keepdims=True)
    scale = jnp.float32(1.0) / jnp.maximum(jnp.sqrt(ss), jnp.float32(1e-12))
    o_ref[...] = w * scale


def _normalize_table(weight):
    return pl.pallas_call(
        _tc_norm_body,
        out_shape=jax.ShapeDtypeStruct((VOCAB, D), jnp.float32),
        grid=(VOCAB // TC_BLOCK,),
        in_specs=[pl.BlockSpec((TC_BLOCK, D), lambda i: (i, 0))],
        out_specs=pl.BlockSpec((TC_BLOCK, D), lambda i: (i, 0)),
        compiler_params=pltpu.CompilerParams(
            dimension_semantics=("parallel",)
        ),
    )(weight)


def _sc_body(x_hbm, w_hbm, out_hbm, idx_v, *bufs_and_sems):
    bufs = bufs_and_sems[:NSLOT]
    gsem = bufs_and_sems[NSLOT:2 * NSLOT]
    ssem = bufs_and_sems[2 * NSLOT:3 * NSLOT]

    c = lax.axis_index("c")
    s = lax.axis_index("s")
    wid = s * NC + c

    pltpu.sync_copy(x_hbm.at[wid], idx_v)

    def gather(j, buf, sem):
        return pltpu.async_copy(w_hbm.at[idx_v.at[j]], buf, sem)

    def wait_gather(j, buf, sem):
        pltpu.make_async_copy(w_hbm.at[idx_v.at[j]], buf, sem).wait()

    def store(j, buf, sem):
        base = (wid * NCHUNK + j) * CHUNK
        return pltpu.async_copy(buf, out_hbm.at[pl.ds(base, CHUNK)], sem)

    def wait_store(j, buf, sem):
        base = (wid * NCHUNK + j) * CHUNK
        pltpu.make_async_copy(buf, out_hbm.at[pl.ds(base, CHUNK)], sem).wait()

    for k in range(NSLOT):
        gather(k, bufs[k], gsem[k])

    def body(i, carry):
        j0 = NSLOT * i
        for k in range(NSLOT):
            wait_gather(j0 + k, bufs[k], gsem[k])
            store(j0 + k, bufs[k], ssem[k])
        for k in range(NSLOT):
            wait_store(j0 + k, bufs[k], ssem[k])
            gather(j0 + NSLOT + k, bufs[k], gsem[k])
        return carry

    lax.fori_loop(0, NCHUNK // NSLOT - 1, body, 0, unroll=False)

    j0 = NCHUNK - NSLOT
    hs = []
    for k in range(NSLOT):
        wait_gather(j0 + k, bufs[k], gsem[k])
        hs.append(store(j0 + k, bufs[k], ssem[k]))
    for h in hs:
        h.wait()


@jax.jit
def kernel(x, weight):
    wn = _normalize_table(weight)
    xi = x.astype(jnp.int32).T.reshape(NW, NCHUNK, CHUNK)
    mesh = plsc.VectorSubcoreMesh(core_axis_name="c", subcore_axis_name="s")
    out = pl.kernel(
        _sc_body,
        out_type=jax.ShapeDtypeStruct((ROWS, D), jnp.float32),
        mesh=mesh,
        compiler_params=pltpu.CompilerParams(needs_layout_passes=False),
        scratch_types=(
            [pltpu.VMEM((NCHUNK, CHUNK), jnp.int32)]
            + [pltpu.VMEM((CHUNK, D), jnp.float32) for _ in range(NSLOT)]
            + [pltpu.SemaphoreType.DMA for _ in range(2 * NSLOT)]
        ),
    )(xi, wn)
    return out.reshape(HIST, BATCH, D).swapaxes(0, 1)


# speedup vs baseline: 7.9797x; 1.0084x over previous
import functools

import jax
import jax.numpy as jnp
from jax import lax
from jax.experimental import pallas as pl
from jax.experimental.pallas import tpu as pltpu
from jax.experimental.pallas import tpu_sc as plsc

VOCAB = 100000
D = 128
BATCH = 4096
HIST = 50

NC = 2
NS = 16
NW = NC * NS
ROWS = BATCH * HIST
R_PER_W = ROWS // NW
CHUNK = 128
NCHUNK = R_PER_W // CHUNK
NSLOT = 7
NFULL = (NCHUNK - NSLOT) // NSLOT
NREM = NCHUNK - NSLOT * (NFULL + 1)

TC_BLOCK = 10000


def _tc_norm_body(w_ref, o_ref):
    w = w_ref[...]
    ss = jnp.sum(w * w, axis=1, keepdims=True)
    scale = jnp.float32(1.0) / jnp.maximum(jnp.sqrt(ss), jnp.float32(1e-12))
    o_ref[...] = w * scale


def _normalize_table(weight):
    return pl.pallas_call(
        _tc_norm_body,
        out_shape=jax.ShapeDtypeStruct((VOCAB, D), jnp.float32),
        grid=(VOCAB // TC_BLOCK,),
        in_specs=[pl.BlockSpec((TC_BLOCK, D), lambda i: (i, 0))],
        out_specs=pl.BlockSpec((TC_BLOCK, D), lambda i: (i, 0)),
        compiler_params=pltpu.CompilerParams(
            dimension_semantics=("parallel",)
        ),
    )(weight)


def _sc_body(x_hbm, w_hbm, out_hbm, idx_v, *bufs_and_sems):
    bufs = bufs_and_sems[:NSLOT]
    gsem = bufs_and_sems[NSLOT:2 * NSLOT]
    ssem = bufs_and_sems[2 * NSLOT:3 * NSLOT]

    c = lax.axis_index("c")
    s = lax.axis_index("s")
    wid = s * NC + c

    pltpu.sync_copy(x_hbm.at[wid], idx_v)

    def gather(j, buf, sem):
        return pltpu.async_copy(w_hbm.at[idx_v.at[j]], buf, sem)

    def wait_gather(j, buf, sem):
        pltpu.make_async_copy(w_hbm.at[idx_v.at[j]], buf, sem).wait()

    def store(j, buf, sem):
        base = (wid * NCHUNK + j) * CHUNK
        return pltpu.async_copy(buf, out_hbm.at[pl.ds(base, CHUNK)], sem)

    def wait_store(j, buf, sem):
        base = (wid * NCHUNK + j) * CHUNK
        pltpu.make_async_copy(buf, out_hbm.at[pl.ds(base, CHUNK)], sem).wait()

    for k in range(NSLOT):
        gather(k, bufs[k], gsem[k])

    def body(i, carry):
        j0 = NSLOT * i
        for k in range(NSLOT):
            wait_gather(j0 + k, bufs[k], gsem[k])
            store(j0 + k, bufs[k], ssem[k])
        for k in range(NSLOT):
            wait_store(j0 + k, bufs[k], ssem[k])
            gather(j0 + NSLOT + k, bufs[k], gsem[k])
        return carry

    lax.fori_loop(0, NFULL, body, 0, unroll=False)

    j0 = NSLOT * NFULL
    for k in range(NSLOT):
        wait_gather(j0 + k, bufs[k], gsem[k])
        store(j0 + k, bufs[k], ssem[k])
    for k in range(NREM):
        wait_store(j0 + k, bufs[k], ssem[k])
        gather(j0 + NSLOT + k, bufs[k], gsem[k])
    for k in range(NREM, NSLOT):
        wait_store(j0 + k, bufs[k], ssem[k])
    j1 = j0 + NSLOT
    hs = []
    for k in range(NREM):
        wait_gather(j1 + k, bufs[k], gsem[k])
        hs.append(store(j1 + k, bufs[k], ssem[k]))
    for h in hs:
        h.wait()


@jax.jit
def kernel(x, weight):
    wn = _normalize_table(weight)
    xi = x.astype(jnp.int32).T.reshape(NW, NCHUNK, CHUNK)
    mesh = plsc.VectorSubcoreMesh(core_axis_name="c", subcore_axis_name="s")
    out = pl.kernel(
        _sc_body,
        out_type=jax.ShapeDtypeStruct((ROWS, D), jnp.float32),
        mesh=mesh,
        compiler_params=pltpu.CompilerParams(needs_layout_passes=False),
        scratch_types=(
            [pltpu.VMEM((NCHUNK, CHUNK), jnp.int32)]
            + [pltpu.VMEM((CHUNK, D), jnp.float32) for _ in range(NSLOT)]
            + [pltpu.SemaphoreType.DMA for _ in range(2 * NSLOT)]
        ),
    )(xi, wn)
    return out.reshape(HIST, BATCH, D).swapaxes(0, 1)
